# baseline (speedup 1.0000x reference)
import jax
import jax.numpy as jnp
from jax.experimental import pallas as pl
from jax.experimental.pallas import tpu as pltpu

_NCH = 8
_NBUF = 4


def _dma_copy(w_ref, o_ref, *rest):
    bufs = rest[:_NBUF]
    sem_in, sem_out = rest[_NBUF], rest[_NBUF + 1]
    ch = o_ref.shape[0] // _NCH

    def in_copy(c):
        return pltpu.make_async_copy(
            w_ref.at[pl.ds(c * ch, ch)], bufs[c % _NBUF], sem_in)

    def out_copy(c):
        return pltpu.make_async_copy(
            bufs[c % _NBUF], o_ref.at[pl.ds(c * ch, ch)], sem_out)

    for c in range(_NBUF):
        in_copy(c).start()
    for c in range(_NCH):
        in_copy(c).wait()
        out_copy(c).start()
        if c + _NBUF < _NCH:
            out_copy(c).wait()
            in_copy(c + _NBUF).start()
    for c in range(_NCH - _NBUF, _NCH):
        out_copy(c).wait()


def kernel(input_ids, weights):
    seq_len = input_ids.shape[-1]
    dim = weights.shape[1]
    assert seq_len % _NCH == 0
    ch = seq_len // _NCH
    return pl.pallas_call(
        _dma_copy,
        in_specs=[pl.BlockSpec(memory_space=pltpu.MemorySpace.HBM)],
        out_specs=pl.BlockSpec(memory_space=pltpu.MemorySpace.HBM),
        out_shape=jax.ShapeDtypeStruct((seq_len, dim), weights.dtype),
        scratch_shapes=[pltpu.VMEM((ch, dim), jnp.float32) for _ in range(_NBUF)]
                       + [pltpu.SemaphoreType.DMA, pltpu.SemaphoreType.DMA],
    )(weights)
